# Initial kernel scaffold; baseline (speedup 1.0000x reference)
import jax
import jax.numpy as jnp
from jax.experimental import pallas as pl


def kernel(seq_len, table):
    raise NotImplementedError("write your pallas kernel here")


# speedup vs baseline: 1.3941x; 1.3941x over previous
import functools

import jax
import jax.numpy as jnp
from jax import lax
from jax.experimental import pallas as pl
from jax.experimental.pallas import tpu as pltpu
from jax.experimental.pallas import tpu_sc as plsc

MAX_LEN = 8192
DIM = 1024

_info = plsc.get_sparse_core_info()
_NC, _NS, _L = _info.num_cores, _info.num_subcores, _info.num_lanes
_NW = _NC * _NS
_ROWS_PER_W = MAX_LEN // _NW
_CHUNK = 32
_NCHUNK = _ROWS_PER_W // _CHUNK


def _pe_kernel(clamp_hbm, table_hbm, out_hbm, clamp_v, idx_v, rows_v, sem, osem):
    wid = lax.axis_index("s") * _NC + lax.axis_index("c")
    base = wid * _ROWS_PER_W

    pltpu.sync_copy(clamp_hbm, clamp_v)
    clamp_vec = clamp_v[...]
    iota = lax.iota(jnp.int32, _L)

    def chunk_body(c, _):
        row0 = base + c * _CHUNK
        for j in range(_CHUNK // _L):
            idx_v[pl.ds(j * _L, _L)] = jnp.minimum(iota + (row0 + j * _L),
                                                   clamp_vec)
        pltpu.async_copy(table_hbm.at[idx_v], rows_v, sem).wait()
        pltpu.async_copy(rows_v, out_hbm.at[pl.ds(row0, _CHUNK)], osem).wait()
        return ()

    lax.fori_loop(0, _NCHUNK, chunk_body, ())


@functools.partial(
    pl.kernel,
    out_type=jax.ShapeDtypeStruct((MAX_LEN, DIM), jnp.float32),
    mesh=plsc.VectorSubcoreMesh(core_axis_name="c", subcore_axis_name="s"),
    scratch_types=[
        pltpu.VMEM((_L,), jnp.int32),
        pltpu.VMEM((_CHUNK,), jnp.int32),
        pltpu.VMEM((_CHUNK, DIM), jnp.float32),
        pltpu.SemaphoreType.DMA,
        pltpu.SemaphoreType.DMA,
    ],
)
def _pe_call(clamp_hbm, table_hbm, out_hbm, clamp_v, idx_v, rows_v, sem, osem):
    _pe_kernel(clamp_hbm, table_hbm, out_hbm, clamp_v, idx_v, rows_v, sem, osem)


def kernel(seq_len, table):
    clamp = jnp.full((_L,), jnp.asarray(seq_len, jnp.int32) - 1, jnp.int32)
    return _pe_call(clamp, table)
